# baseline (speedup 1.0000x reference)
import functools

import jax
import jax.numpy as jnp
import numpy as np
from jax import lax
from jax.experimental import pallas as pl
from jax.experimental.pallas import tpu as pltpu
from jax.experimental.pallas import tpu_sc as plsc

B, C, H, W = 4, 256, 256, 256
TOPK_GEO = 20
TOPK_STATIC = 15
NUM_AMB = 30
TEMP = 0.1
MARGIN = 0.3
PCR0, PCR1 = -59.9, -59.9
BEGIN_W = 119.8
BEGIN_H = 119.8
NEG_INF = float("-inf")

CB = 32
NC = C // CB

_AMB_IDX = np.array([
    [42684, 39799, 17101, 26684, 45502, 49427, 52287, 15840, 51402, 10298,
     15388, 5700, 62806, 48744, 54950, 1748, 30316, 65134, 23053, 13198,
     40558, 26643, 43274, 16075, 23612, 1587, 29516, 8934, 60384, 4667],
    [8159, 21315, 62515, 17158, 55309, 59725, 2179, 36669, 1658, 62896,
     9308, 39123, 42076, 23448, 35406, 13534, 51694, 37155, 53091, 24021,
     19512, 28969, 18536, 8640, 62520, 51289, 1823, 10367, 53219, 3871],
    [11764, 16683, 42054, 43176, 18925, 52470, 33349, 61666, 38626, 7651,
     8719, 61782, 25143, 31664, 42585, 38596, 65445, 34154, 60634, 18834,
     37711, 21348, 30494, 37663, 56142, 1625, 24231, 53445, 7680, 31046],
    [1614, 28512, 63057, 22698, 20162, 35211, 47616, 17187, 64995, 28178,
     36068, 40764, 24733, 1009, 15346, 26315, 6708, 34707, 37933, 19975,
     47948, 45082, 17151, 32451, 5007, 5526, 40990, 1517, 9641, 1470]],
    dtype=np.int32)

_STATIC_PERM = np.array([
    [35, 6, 30, 28, 33, 1, 2, 37, 45, 17, 32, 48, 14, 4, 7],
    [11, 3, 18, 10, 44, 16, 21, 48, 17, 35, 2, 23, 33, 43, 0],
    [14, 13, 15, 7, 17, 11, 32, 44, 35, 1, 36, 28, 31, 2, 9],
    [28, 0, 47, 27, 48, 45, 29, 15, 1, 25, 16, 13, 42, 46, 19]],
    dtype=np.int32)
_STATIC_PERM16 = np.concatenate(
    [_STATIC_PERM, np.full((4, 1), -1, np.int32)], axis=1)


def _phase_a_body(perm_ref, gt_ref, x_ref, geo_ref, gv_ref, st_ref, acc_ref):
    c = pl.program_id(1)
    blk = x_ref[0]
    part = jnp.sum(blk * blk, axis=0)

    @pl.when(c == 0)
    def _init():
        acc_ref[...] = part

    @pl.when(c > 0)
    def _acc():
        acc_ref[...] = acc_ref[...] + part

    @pl.when(c == NC - 1)
    def _finish():
        inten = jnp.sqrt(acc_ref[...])
        ninf = jnp.float32(NEG_INF)
        pad_row = jnp.full((1, W), ninf, jnp.float32)
        up = jnp.concatenate([inten[1:, :], pad_row], axis=0)
        dn = jnp.concatenate([pad_row, inten[:-1, :]], axis=0)
        rowm = jnp.maximum(inten, jnp.maximum(up, dn))
        pad_col = jnp.full((H, 1), ninf, jnp.float32)
        lf = jnp.concatenate([rowm[:, 1:], pad_col], axis=1)
        rt = jnp.concatenate([pad_col, rowm[:, :-1]], axis=1)
        pooled = jnp.maximum(rowm, jnp.maximum(lf, rt))
        mask = inten == pooled
        cand0 = jnp.where(mask, inten, ninf)
        flat = (lax.broadcasted_iota(jnp.int32, (H, W), 0) * W
                + lax.broadcasted_iota(jnp.int32, (H, W), 1))
        lane = lax.broadcasted_iota(jnp.int32, (1, 128), 1)

        def topk_step(k, carry):
            cand, idxv, gvv = carry
            m = jnp.max(cand)
            sel = jnp.where(cand == m, flat, jnp.int32(1 << 30))
            i = jnp.min(sel)
            idxv = jnp.where(lane == k, i, idxv)
            gvv = jnp.where(lane == k,
                            jnp.where(m > ninf, 1, 0), gvv)
            cand = jnp.where(flat == i, ninf, cand)
            return cand, idxv, gvv

        _, idxv, gvv = lax.fori_loop(
            0, TOPK_GEO, topk_step,
            (cand0, jnp.zeros((1, 128), jnp.int32),
             jnp.zeros((1, 128), jnp.int32)))
        geo_ref[0] = idxv
        gv_ref[0] = gvv

        boxes = gt_ref[0]
        n = boxes.shape[0]
        kcol = (boxes[:, 7:8] != 0).astype(jnp.float32)
        tri = (lax.broadcasted_iota(jnp.int32, (n, n), 0)
               > lax.broadcasted_iota(jnp.int32, (n, n), 1)
               ).astype(jnp.float32)
        ones_before = jnp.dot(tri, kcol,
                              preferred_element_type=jnp.float32)
        zeros_before = jnp.dot(tri, 1.0 - kcol,
                               preferred_element_type=jnp.float32)
        nz = jnp.sum(1.0 - kcol)
        pos = jnp.where(kcol > 0, nz + ones_before, zeros_before)
        targets = perm_ref[0].astype(jnp.float32)
        eq = (pos == targets).astype(jnp.float32)
        bx = jnp.sum(eq * boxes[:, 0:1], axis=0, keepdims=True)
        by = jnp.sum(eq * boxes[:, 1:2], axis=0, keepdims=True)
        cx = jnp.clip((bx - PCR0) / BEGIN_W * W, 0, W - 1).astype(jnp.int32)
        cy = jnp.clip((by - PCR1) / BEGIN_H * H, 0, H - 1).astype(jnp.int32)
        st_ref[0] = cy * W + cx


def _phase_a(x, gt_boxes):
    perm = jnp.asarray(_STATIC_PERM16).reshape(4, 1, 16)
    grid = (B, NC)
    return pl.pallas_call(
        _phase_a_body,
        grid=grid,
        in_specs=[
            pl.BlockSpec((1, 1, 16), lambda b, c: (b, 0, 0)),
            pl.BlockSpec((1, 50, 8), lambda b, c: (b, 0, 0)),
            pl.BlockSpec((1, CB, H, W), lambda b, c: (b, c, 0, 0)),
        ],
        out_specs=[
            pl.BlockSpec((1, 1, 128), lambda b, c: (b, 0, 0)),
            pl.BlockSpec((1, 1, 128), lambda b, c: (b, 0, 0)),
            pl.BlockSpec((1, 1, 16), lambda b, c: (b, 0, 0)),
        ],
        out_shape=[
            jax.ShapeDtypeStruct((B, 1, 128), jnp.int32),
            jax.ShapeDtypeStruct((B, 1, 128), jnp.int32),
            jax.ShapeDtypeStruct((B, 1, 16), jnp.int32),
        ],
        scratch_shapes=[pltpu.VMEM((H, W), jnp.float32)],
        compiler_params=pltpu.CompilerParams(
            dimension_semantics=("arbitrary", "arbitrary")),
    )(perm, gt_boxes, x)


NV = 80 + 60 + 120
VPT = 17
N_GEO = 80
N_ST = 60
N_AMB = 120
AMB_PER_SC = 60
ROWS_PER_TILE = 4
TROWS = 3 * 16 * VPT
BIAS_INVALID = -1e37


_SLOT_MAP = np.minimum(
    (np.arange(16)[:, None] * VPT + np.arange(32)[None, :]), NV - 1
).astype(np.int32)


def _phase_b(x1d, base_tbl, bias_tbl):
    mesh = plsc.VectorSubcoreMesh(core_axis_name="c", subcore_axis_name="s")

    @functools.partial(
        pl.kernel,
        out_type=jax.ShapeDtypeStruct((32, 16), jnp.float32),
        mesh=mesh,
        compiler_params=pltpu.CompilerParams(needs_layout_passes=False),
        scratch_types=[
            pltpu.VMEM((32,), jnp.int32),
            pltpu.VMEM((32,), jnp.float32),
            pltpu.VMEM((2 * VPT, 128), jnp.int32),
            pltpu.VMEM((3 * VPT, 128), jnp.float32),
            pltpu.VMEM((3 * (N_GEO + N_ST), 128), jnp.float32),
            pltpu.VMEM((3 * ROWS_PER_TILE, 128), jnp.float32),
            pltpu.VMEM((16,), jnp.float32),
            pltpu.VMEM_SHARED((TROWS, 128), jnp.float32),
            pltpu.SemaphoreType.DMA,
        ],
    )
    def body(x_hbm, base_hbm, bias_hbm, out_hbm,
             base_v, bias_v, gidx, gbuf, tgt, myamb, obuf, table, sem):
        s = lax.axis_index("s")
        c = lax.axis_index("c")
        wid = s * 2 + c

        pltpu.sync_copy(base_hbm.at[s], base_v)
        pltpu.sync_copy(bias_hbm.at[s], bias_v)
        iota16 = lax.broadcasted_iota(jnp.int32, (16,), 0)
        zeros16 = jnp.zeros((16,), jnp.float32)
        v0 = s * VPT
        bvecs = [base_v[pl.ds(0, 16)], base_v[pl.ds(16, 16)]]
        fvecs = [bias_v[pl.ds(0, 16)], bias_v[pl.ds(16, 16)]]

        for i in range(VPT):
            base = bvecs[i // 16][i % 16]
            for q in range(16):
                gidx[2 * i + q // 8, pl.ds((q % 8) * 16, 16)] = (
                    base + (H * W) * (q * 16 + iota16))

        def _fire(r, carry):
            dst = 3 * (r >> 1) + (r & 1)
            pltpu.async_copy(x_hbm.at[gidx.at[r]], gbuf.at[dst], sem)
            return carry

        lax.fori_loop(0, 2 * VPT, _fire, 0)

        def _drain(r, carry):
            dst = 3 * (r >> 1) + (r & 1)
            pltpu.make_async_copy(
                x_hbm.at[gidx.at[r]], gbuf.at[dst], sem).wait()
            return carry

        lax.fori_loop(0, 2 * VPT, _drain, 0)

        for i in range(VPT):
            chunks = []
            acc = None
            for q in range(16):
                ch = gbuf[3 * i + q // 8, pl.ds((q % 8) * 16, 16)]
                chunks.append(ch)
                acc = ch * ch if acc is None else acc + ch * ch
            ssq = jnp.sum(acc)
            sv = jnp.broadcast_to(ssq, (16,))
            yi = jnp.int32(0x5F3759DF) - (plsc.bitcast(sv, jnp.int32) >> 1)
            y = plsc.bitcast(yi, jnp.float32)
            for _ in range(3):
                y = y * (1.5 - 0.5 * sv * y * y)
            inv = jnp.where(sv > 1e-24, y, jnp.full((16,), 1e12, jnp.float32))
            for q in range(16):
                gbuf[3 * i + q // 8, pl.ds((q % 8) * 16, 16)] = (
                    chunks[q] * inv)
            brow = fvecs[i // 16][i % 16]
            gbuf[3 * i + 2, pl.ds(0, 16)] = jnp.where(
                iota16 == 0, brow, zeros16)
            for q in range(1, 8):
                gbuf[3 * i + 2, pl.ds(q * 16, 16)] = zeros16

        pltpu.sync_copy(gbuf, table.at[pl.ds(v0 * 3, 3 * VPT)])
        plsc.subcore_barrier()
        pltpu.sync_copy(table.at[pl.ds(0, 3 * (N_GEO + N_ST))], tgt)
        amb_row0 = N_GEO + N_ST + c * AMB_PER_SC + s * ROWS_PER_TILE
        pltpu.sync_copy(
            table.at[pl.ds(amb_row0 * 3, 3 * ROWS_PER_TILE)], myamb)

        total = jnp.float32(0.0)
        inv_temp = jnp.float32(float(np.float32(1.0) / np.float32(TEMP)))
        for j in range(ROWS_PER_TILE):
            arow = [myamb[3 * j + q // 8, pl.ds((q % 8) * 16, 16)]
                    for q in range(16)]
            arow.append(myamb[3 * j + 2, pl.ds(0, 16)])

            def _sim(t):
                acc = None
                for q in range(17):
                    row = 3 * t + (q // 8 if q < 16 else 2)
                    off = (q % 8) * 16 if q < 16 else 0
                    tc = tgt[row, pl.ds(off, 16)]
                    acc = arow[q] * tc if acc is None else acc + arow[q] * tc
                return jnp.sum(acc) * inv_temp

            def _max(t, m):
                return jnp.maximum(m, _sim(t))

            mp = lax.fori_loop(0, N_GEO, _max, jnp.float32(-3.4e38))
            mn = lax.fori_loop(N_GEO, N_GEO + N_ST, _max,
                               jnp.float32(-3.4e38))
            term = jnp.maximum(jnp.float32(MARGIN) + mn - mp,
                               jnp.float32(0.0))
            in_range = s * ROWS_PER_TILE + j < AMB_PER_SC
            total = total + jnp.where(in_range, term, jnp.float32(0.0))

        obuf[...] = jnp.broadcast_to(total, (16,))
        pltpu.sync_copy(obuf, out_hbm.at[wid])

    return body(x1d, base_tbl, bias_tbl)


def kernel(aligned_loc_feature, gt_boxes):
    geo, gv, st = _phase_a(aligned_loc_feature, gt_boxes)
    x1d = aligned_loc_feature.reshape(-1)
    geo2 = geo.reshape(B, 128)[:, :TOPK_GEO]
    gv2 = gv.reshape(B, 128)[:, :TOPK_GEO]
    st2 = st.reshape(B, 16)[:, :TOPK_STATIC]
    amb2 = jnp.asarray(_AMB_IDX)
    bvals = jnp.concatenate([
        jnp.repeat(jnp.arange(B, dtype=jnp.int32), TOPK_GEO),
        jnp.repeat(jnp.arange(B, dtype=jnp.int32), TOPK_STATIC),
        jnp.repeat(jnp.arange(B, dtype=jnp.int32), NUM_AMB)])
    pvals = jnp.concatenate(
        [geo2.reshape(-1), st2.reshape(-1), amb2.reshape(-1)])
    base_flat = bvals * (C * H * W) + pvals
    bias_flat = jnp.concatenate([
        jnp.where(gv2.reshape(-1) != 0, 0.0, BIAS_INVALID)
        .astype(jnp.float32),
        jnp.zeros((N_ST,), jnp.float32),
        jnp.ones((N_AMB,), jnp.float32)])
    slot = jnp.asarray(_SLOT_MAP)
    partials = _phase_b(x1d, base_flat[slot], bias_flat[slot])
    return jnp.sum(partials[:, 0]) / jnp.float32(N_AMB)


# speedup vs baseline: 1.6381x; 1.6381x over previous
import functools

import jax
import jax.numpy as jnp
import numpy as np
from jax import lax
from jax.experimental import pallas as pl
from jax.experimental.pallas import tpu as pltpu
from jax.experimental.pallas import tpu_sc as plsc

B, C, H, W = 4, 256, 256, 256
TOPK_GEO = 20
TOPK_STATIC = 15
NUM_AMB = 30
TEMP = 0.1
MARGIN = 0.3
PCR0, PCR1 = -59.9, -59.9
BEGIN_W = 119.8
BEGIN_H = 119.8
NEG_INF = float("-inf")

CB = 32
NC = C // CB

_AMB_IDX = np.array([
    [42684, 39799, 17101, 26684, 45502, 49427, 52287, 15840, 51402, 10298,
     15388, 5700, 62806, 48744, 54950, 1748, 30316, 65134, 23053, 13198,
     40558, 26643, 43274, 16075, 23612, 1587, 29516, 8934, 60384, 4667],
    [8159, 21315, 62515, 17158, 55309, 59725, 2179, 36669, 1658, 62896,
     9308, 39123, 42076, 23448, 35406, 13534, 51694, 37155, 53091, 24021,
     19512, 28969, 18536, 8640, 62520, 51289, 1823, 10367, 53219, 3871],
    [11764, 16683, 42054, 43176, 18925, 52470, 33349, 61666, 38626, 7651,
     8719, 61782, 25143, 31664, 42585, 38596, 65445, 34154, 60634, 18834,
     37711, 21348, 30494, 37663, 56142, 1625, 24231, 53445, 7680, 31046],
    [1614, 28512, 63057, 22698, 20162, 35211, 47616, 17187, 64995, 28178,
     36068, 40764, 24733, 1009, 15346, 26315, 6708, 34707, 37933, 19975,
     47948, 45082, 17151, 32451, 5007, 5526, 40990, 1517, 9641, 1470]],
    dtype=np.int32)

_STATIC_PERM = np.array([
    [35, 6, 30, 28, 33, 1, 2, 37, 45, 17, 32, 48, 14, 4, 7],
    [11, 3, 18, 10, 44, 16, 21, 48, 17, 35, 2, 23, 33, 43, 0],
    [14, 13, 15, 7, 17, 11, 32, 44, 35, 1, 36, 28, 31, 2, 9],
    [28, 0, 47, 27, 48, 45, 29, 15, 1, 25, 16, 13, 42, 46, 19]],
    dtype=np.int32)
_STATIC_PERM16 = np.concatenate(
    [_STATIC_PERM, np.full((4, 1), -1, np.int32)], axis=1)


def _phase_a_body(perm_ref, gt_ref, x_ref, geo_ref, gv_ref, st_ref, acc_ref):
    c = pl.program_id(1)
    blk = x_ref[0]
    part = jnp.sum(blk * blk, axis=0)

    @pl.when(c == 0)
    def _init():
        acc_ref[...] = part

    @pl.when(c > 0)
    def _acc():
        acc_ref[...] = acc_ref[...] + part

    @pl.when(c == NC - 1)
    def _finish():
        inten = jnp.sqrt(acc_ref[...])
        ninf = jnp.float32(NEG_INF)
        pad_row = jnp.full((1, W), ninf, jnp.float32)
        up = jnp.concatenate([inten[1:, :], pad_row], axis=0)
        dn = jnp.concatenate([pad_row, inten[:-1, :]], axis=0)
        rowm = jnp.maximum(inten, jnp.maximum(up, dn))
        pad_col = jnp.full((H, 1), ninf, jnp.float32)
        lf = jnp.concatenate([rowm[:, 1:], pad_col], axis=1)
        rt = jnp.concatenate([pad_col, rowm[:, :-1]], axis=1)
        pooled = jnp.maximum(rowm, jnp.maximum(lf, rt))
        mask = inten == pooled
        cand0 = jnp.where(mask, inten, ninf)
        flat = (lax.broadcasted_iota(jnp.int32, (H, W), 0) * W
                + lax.broadcasted_iota(jnp.int32, (H, W), 1))
        lane = lax.broadcasted_iota(jnp.int32, (1, 128), 1)

        def topk_step(k, carry):
            cand, idxv, gvv = carry
            m = jnp.max(cand)
            sel = jnp.where(cand == m, flat, jnp.int32(1 << 30))
            i = jnp.min(sel)
            idxv = jnp.where(lane == k, i, idxv)
            gvv = jnp.where(lane == k,
                            jnp.where(m > ninf, 1, 0), gvv)
            cand = jnp.where(flat == i, ninf, cand)
            return cand, idxv, gvv

        _, idxv, gvv = lax.fori_loop(
            0, TOPK_GEO, topk_step,
            (cand0, jnp.zeros((1, 128), jnp.int32),
             jnp.zeros((1, 128), jnp.int32)))
        geo_ref[0] = idxv
        gv_ref[0] = gvv

        boxes = gt_ref[0]
        n = boxes.shape[0]
        kcol = (boxes[:, 7:8] != 0).astype(jnp.float32)
        tri = (lax.broadcasted_iota(jnp.int32, (n, n), 0)
               > lax.broadcasted_iota(jnp.int32, (n, n), 1)
               ).astype(jnp.float32)
        ones_before = jnp.dot(tri, kcol,
                              preferred_element_type=jnp.float32)
        zeros_before = jnp.dot(tri, 1.0 - kcol,
                               preferred_element_type=jnp.float32)
        nz = jnp.sum(1.0 - kcol)
        pos = jnp.where(kcol > 0, nz + ones_before, zeros_before)
        targets = perm_ref[0].astype(jnp.float32)
        eq = (pos == targets).astype(jnp.float32)
        bx = jnp.sum(eq * boxes[:, 0:1], axis=0, keepdims=True)
        by = jnp.sum(eq * boxes[:, 1:2], axis=0, keepdims=True)
        cx = jnp.clip((bx - PCR0) / BEGIN_W * W, 0, W - 1).astype(jnp.int32)
        cy = jnp.clip((by - PCR1) / BEGIN_H * H, 0, H - 1).astype(jnp.int32)
        st_ref[0] = cy * W + cx


def _phase_a(x, gt_boxes):
    perm = jnp.asarray(_STATIC_PERM16).reshape(4, 1, 16)
    grid = (B, NC)
    return pl.pallas_call(
        _phase_a_body,
        grid=grid,
        in_specs=[
            pl.BlockSpec((1, 1, 16), lambda b, c: (b, 0, 0)),
            pl.BlockSpec((1, 50, 8), lambda b, c: (b, 0, 0)),
            pl.BlockSpec((1, CB, H, W), lambda b, c: (b, c, 0, 0)),
        ],
        out_specs=[
            pl.BlockSpec((1, 1, 128), lambda b, c: (b, 0, 0)),
            pl.BlockSpec((1, 1, 128), lambda b, c: (b, 0, 0)),
            pl.BlockSpec((1, 1, 16), lambda b, c: (b, 0, 0)),
        ],
        out_shape=[
            jax.ShapeDtypeStruct((B, 1, 128), jnp.int32),
            jax.ShapeDtypeStruct((B, 1, 128), jnp.int32),
            jax.ShapeDtypeStruct((B, 1, 16), jnp.int32),
        ],
        scratch_shapes=[pltpu.VMEM((H, W), jnp.float32)],
        compiler_params=pltpu.CompilerParams(
            dimension_semantics=("arbitrary", "arbitrary")),
    )(perm, gt_boxes, x)


NV = 80 + 60 + 120
VPT = 17
N_GEO = 80
N_ST = 60
N_AMB = 120
AMB_PER_SC = 60
ROWS_PER_TILE = 4
TROWS = 3 * 16 * VPT
BIAS_INVALID = -1e37


_SLOT_MAP = np.minimum(
    (np.arange(16)[:, None] * VPT + np.arange(32)[None, :]), NV - 1
).astype(np.int32)


def _phase_b(x1d, base_tbl, xpos_tbl, bias_tbl):
    mesh = plsc.VectorSubcoreMesh(core_axis_name="c", subcore_axis_name="s")

    @functools.partial(
        pl.kernel,
        out_type=jax.ShapeDtypeStruct((32, 16), jnp.float32),
        mesh=mesh,
        compiler_params=pltpu.CompilerParams(
            needs_layout_passes=False, use_tc_tiling_on_sc=True),
        scratch_types=[
            pltpu.VMEM((32,), jnp.int32),
            pltpu.VMEM((32,), jnp.int32),
            pltpu.VMEM((32,), jnp.float32),
            pltpu.VMEM((2 * VPT, 128), jnp.int32),
            pltpu.VMEM((3 * VPT, 128), jnp.float32),
            pltpu.VMEM((2, 128, 128), jnp.float32),
            pltpu.VMEM((3 * (N_GEO + N_ST), 128), jnp.float32),
            pltpu.VMEM((3 * ROWS_PER_TILE, 128), jnp.float32),
            pltpu.VMEM((16,), jnp.float32),
            pltpu.VMEM_SHARED((TROWS, 128), jnp.float32),
            pltpu.SMEM((24,), jnp.int32),
            pltpu.SMEM((24,), jnp.int32),
            pltpu.SMEM((24,), jnp.float32),
            pltpu.SemaphoreType.DMA,
        ],
    )
    def body(x_hbm, base_hbm, xpos_hbm, bias_hbm, out_hbm,
             base_v, xpos_v, bias_v, gidx, gbuf, rawbuf, tgt, myamb, obuf,
             table, sm_base, sm_x, sm_bias, sem):
        x2 = x_hbm.reshape(B * C * H, W)
        s = lax.axis_index("s")
        c = lax.axis_index("c")
        wid = s * 2 + c

        pltpu.sync_copy(base_hbm.at[s], base_v)
        pltpu.sync_copy(bias_hbm.at[s], bias_v)
        pltpu.sync_copy(xpos_hbm.at[s], xpos_v)
        iota16 = lax.broadcasted_iota(jnp.int32, (16,), 0)
        zeros16 = jnp.zeros((16,), jnp.float32)
        v0 = s * VPT
        bvecs = [base_v[pl.ds(0, 16)], base_v[pl.ds(16, 16)]]
        xvecs = [xpos_v[pl.ds(0, 16)], xpos_v[pl.ds(16, 16)]]
        fvecs = [bias_v[pl.ds(0, 16)], bias_v[pl.ds(16, 16)]]

        for i in range(VPT):
            sm_base[i] = bvecs[i // 16][i % 16]
            sm_x[i] = xvecs[i // 16][i % 16]
            sm_bias[i] = fvecs[i // 16][i % 16]

        def _build(i, carry):
            base = sm_base[i]
            for q in range(16):
                gidx[2 * i + q // 8, pl.ds((q % 8) * 16, 16)] = (
                    base + H * (q * 16 + iota16))
            return carry

        lax.fori_loop(0, VPT, _build, 0)

        def _fire(hh):
            xa = (sm_x[hh >> 1] >> 7) * 128
            pltpu.async_copy(
                x2.at[gidx.at[hh], pl.ds(xa, 128)], rawbuf.at[hh & 1], sem)

        def _gstep(hh, carry):
            @pl.when(hh + 1 < 2 * VPT)
            def _():
                _fire(hh + 1)

            i = hh >> 1
            half = hh & 1
            xa = (sm_x[i] >> 7) * 128
            pltpu.make_async_copy(
                x2.at[gidx.at[hh], pl.ds(xa, 128)], rawbuf.at[hh & 1],
                sem).wait()
            lane = jnp.broadcast_to(sm_x[i] & 127, (16,))
            rb = rawbuf.at[hh & 1]
            for g in range(8):
                ch = plsc.load_gather(rb, [g * 16 + iota16, lane])
                gbuf[3 * i + half, pl.ds(g * 16, 16)] = ch
            return carry

        _fire(0)
        lax.fori_loop(0, 2 * VPT, _gstep, 0)

        def _norm(i, carry):
            chunks = []
            acc = None
            for q in range(16):
                ch = gbuf[3 * i + q // 8, pl.ds((q % 8) * 16, 16)]
                chunks.append(ch)
                acc = ch * ch if acc is None else acc + ch * ch
            ssq = jnp.sum(acc)
            sv = jnp.broadcast_to(ssq, (16,))
            yi = jnp.int32(0x5F3759DF) - (plsc.bitcast(sv, jnp.int32) >> 1)
            y = plsc.bitcast(yi, jnp.float32)
            for _ in range(3):
                y = y * (1.5 - 0.5 * sv * y * y)
            inv = jnp.where(sv > 1e-24, y, jnp.full((16,), 1e12, jnp.float32))
            for q in range(16):
                gbuf[3 * i + q // 8, pl.ds((q % 8) * 16, 16)] = (
                    chunks[q] * inv)
            brow = sm_bias[i]
            gbuf[3 * i + 2, pl.ds(0, 16)] = jnp.where(
                iota16 == 0, brow, zeros16)
            for q in range(1, 8):
                gbuf[3 * i + 2, pl.ds(q * 16, 16)] = zeros16
            return carry

        lax.fori_loop(0, VPT, _norm, 0)

        pltpu.sync_copy(gbuf, table.at[pl.ds(v0 * 3, 3 * VPT)])
        plsc.subcore_barrier()
        pltpu.sync_copy(table.at[pl.ds(0, 3 * (N_GEO + N_ST))], tgt)
        amb_row0 = N_GEO + N_ST + c * AMB_PER_SC + s * ROWS_PER_TILE
        pltpu.sync_copy(
            table.at[pl.ds(amb_row0 * 3, 3 * ROWS_PER_TILE)], myamb)

        total = jnp.float32(0.0)
        inv_temp = jnp.float32(float(np.float32(1.0) / np.float32(TEMP)))
        for j in range(ROWS_PER_TILE):
            arow = [myamb[3 * j + q // 8, pl.ds((q % 8) * 16, 16)]
                    for q in range(16)]
            arow.append(myamb[3 * j + 2, pl.ds(0, 16)])

            def _sim(t):
                acc = None
                for q in range(17):
                    row = 3 * t + (q // 8 if q < 16 else 2)
                    off = (q % 8) * 16 if q < 16 else 0
                    tc = tgt[row, pl.ds(off, 16)]
                    acc = arow[q] * tc if acc is None else acc + arow[q] * tc
                return jnp.sum(acc) * inv_temp

            def _max(t, m):
                return jnp.maximum(m, _sim(t))

            mp = lax.fori_loop(0, N_GEO, _max, jnp.float32(-3.4e38))
            mn = lax.fori_loop(N_GEO, N_GEO + N_ST, _max,
                               jnp.float32(-3.4e38))
            term = jnp.maximum(jnp.float32(MARGIN) + mn - mp,
                               jnp.float32(0.0))
            in_range = s * ROWS_PER_TILE + j < AMB_PER_SC
            total = total + jnp.where(in_range, term, jnp.float32(0.0))

        obuf[...] = jnp.broadcast_to(total, (16,))
        pltpu.sync_copy(obuf, out_hbm.at[wid])

    return body(x1d, base_tbl, xpos_tbl, bias_tbl)


def kernel(aligned_loc_feature, gt_boxes):
    geo, gv, st = _phase_a(aligned_loc_feature, gt_boxes)
    geo2 = geo.reshape(B, 128)[:, :TOPK_GEO]
    gv2 = gv.reshape(B, 128)[:, :TOPK_GEO]
    st2 = st.reshape(B, 16)[:, :TOPK_STATIC]
    amb2 = jnp.asarray(_AMB_IDX)
    bvals = jnp.concatenate([
        jnp.repeat(jnp.arange(B, dtype=jnp.int32), TOPK_GEO),
        jnp.repeat(jnp.arange(B, dtype=jnp.int32), TOPK_STATIC),
        jnp.repeat(jnp.arange(B, dtype=jnp.int32), NUM_AMB)])
    pvals = jnp.concatenate(
        [geo2.reshape(-1), st2.reshape(-1), amb2.reshape(-1)])
    py, px = pvals >> 8, pvals & 255
    base_flat = bvals * (C * H) + py
    xpos_flat = px
    bias_flat = jnp.concatenate([
        jnp.where(gv2.reshape(-1) != 0, 0.0, BIAS_INVALID)
        .astype(jnp.float32),
        jnp.zeros((N_ST,), jnp.float32),
        jnp.ones((N_AMB,), jnp.float32)])
    slot = jnp.asarray(_SLOT_MAP)
    partials = _phase_b(aligned_loc_feature, base_flat[slot],
                        xpos_flat[slot], bias_flat[slot])
    return jnp.sum(partials[:, 0]) / jnp.float32(N_AMB)


# speedup vs baseline: 1.6381x; 1.0000x over previous
import functools

import jax
import jax.numpy as jnp
import numpy as np
from jax import lax
from jax.experimental import pallas as pl
from jax.experimental.pallas import tpu as pltpu
from jax.experimental.pallas import tpu_sc as plsc

B, C, H, W = 4, 256, 256, 256
TOPK_GEO = 20
TOPK_STATIC = 15
NUM_AMB = 30
TEMP = 0.1
MARGIN = 0.3
PCR0, PCR1 = -59.9, -59.9
BEGIN_W = 119.8
BEGIN_H = 119.8
NEG_INF = float("-inf")

CB = 64
NC = C // CB

_AMB_IDX = np.array([
    [42684, 39799, 17101, 26684, 45502, 49427, 52287, 15840, 51402, 10298,
     15388, 5700, 62806, 48744, 54950, 1748, 30316, 65134, 23053, 13198,
     40558, 26643, 43274, 16075, 23612, 1587, 29516, 8934, 60384, 4667],
    [8159, 21315, 62515, 17158, 55309, 59725, 2179, 36669, 1658, 62896,
     9308, 39123, 42076, 23448, 35406, 13534, 51694, 37155, 53091, 24021,
     19512, 28969, 18536, 8640, 62520, 51289, 1823, 10367, 53219, 3871],
    [11764, 16683, 42054, 43176, 18925, 52470, 33349, 61666, 38626, 7651,
     8719, 61782, 25143, 31664, 42585, 38596, 65445, 34154, 60634, 18834,
     37711, 21348, 30494, 37663, 56142, 1625, 24231, 53445, 7680, 31046],
    [1614, 28512, 63057, 22698, 20162, 35211, 47616, 17187, 64995, 28178,
     36068, 40764, 24733, 1009, 15346, 26315, 6708, 34707, 37933, 19975,
     47948, 45082, 17151, 32451, 5007, 5526, 40990, 1517, 9641, 1470]],
    dtype=np.int32)

_STATIC_PERM = np.array([
    [35, 6, 30, 28, 33, 1, 2, 37, 45, 17, 32, 48, 14, 4, 7],
    [11, 3, 18, 10, 44, 16, 21, 48, 17, 35, 2, 23, 33, 43, 0],
    [14, 13, 15, 7, 17, 11, 32, 44, 35, 1, 36, 28, 31, 2, 9],
    [28, 0, 47, 27, 48, 45, 29, 15, 1, 25, 16, 13, 42, 46, 19]],
    dtype=np.int32)
_STATIC_PERM16 = np.concatenate(
    [_STATIC_PERM, np.full((4, 1), -1, np.int32)], axis=1)


def _phase_a_body(perm_ref, gt_ref, x_ref, geo_ref, gv_ref, st_ref, acc_ref):
    c = pl.program_id(1)
    blk = x_ref[0]
    part = jnp.sum(blk * blk, axis=0)

    @pl.when(c == 0)
    def _init():
        acc_ref[...] = part

    @pl.when(c > 0)
    def _acc():
        acc_ref[...] = acc_ref[...] + part

    @pl.when(c == NC - 1)
    def _finish():
        inten = jnp.sqrt(acc_ref[...])
        ninf = jnp.float32(NEG_INF)
        pad_row = jnp.full((1, W), ninf, jnp.float32)
        up = jnp.concatenate([inten[1:, :], pad_row], axis=0)
        dn = jnp.concatenate([pad_row, inten[:-1, :]], axis=0)
        rowm = jnp.maximum(inten, jnp.maximum(up, dn))
        pad_col = jnp.full((H, 1), ninf, jnp.float32)
        lf = jnp.concatenate([rowm[:, 1:], pad_col], axis=1)
        rt = jnp.concatenate([pad_col, rowm[:, :-1]], axis=1)
        pooled = jnp.maximum(rowm, jnp.maximum(lf, rt))
        mask = inten == pooled
        cand0 = jnp.where(mask, inten, ninf)
        flat = (lax.broadcasted_iota(jnp.int32, (H, W), 0) * W
                + lax.broadcasted_iota(jnp.int32, (H, W), 1))
        lane = lax.broadcasted_iota(jnp.int32, (1, 128), 1)

        def topk_step(k, carry):
            cand, idxv, gvv = carry
            m = jnp.max(cand)
            sel = jnp.where(cand == m, flat, jnp.int32(1 << 30))
            i = jnp.min(sel)
            idxv = jnp.where(lane == k, i, idxv)
            gvv = jnp.where(lane == k,
                            jnp.where(m > ninf, 1, 0), gvv)
            cand = jnp.where(flat == i, ninf, cand)
            return cand, idxv, gvv

        _, idxv, gvv = lax.fori_loop(
            0, TOPK_GEO, topk_step,
            (cand0, jnp.zeros((1, 128), jnp.int32),
             jnp.zeros((1, 128), jnp.int32)))
        geo_ref[0] = idxv
        gv_ref[0] = gvv

        boxes = gt_ref[0]
        n = boxes.shape[0]
        kcol = (boxes[:, 7:8] != 0).astype(jnp.float32)
        tri = (lax.broadcasted_iota(jnp.int32, (n, n), 0)
               > lax.broadcasted_iota(jnp.int32, (n, n), 1)
               ).astype(jnp.float32)
        ones_before = jnp.dot(tri, kcol,
                              preferred_element_type=jnp.float32)
        zeros_before = jnp.dot(tri, 1.0 - kcol,
                               preferred_element_type=jnp.float32)
        nz = jnp.sum(1.0 - kcol)
        pos = jnp.where(kcol > 0, nz + ones_before, zeros_before)
        targets = perm_ref[0].astype(jnp.float32)
        eq = (pos == targets).astype(jnp.float32)
        bx = jnp.sum(eq * boxes[:, 0:1], axis=0, keepdims=True)
        by = jnp.sum(eq * boxes[:, 1:2], axis=0, keepdims=True)
        cx = jnp.clip((bx - PCR0) / BEGIN_W * W, 0, W - 1).astype(jnp.int32)
        cy = jnp.clip((by - PCR1) / BEGIN_H * H, 0, H - 1).astype(jnp.int32)
        st_ref[0] = cy * W + cx


def _phase_a(x, gt_boxes):
    perm = jnp.asarray(_STATIC_PERM16).reshape(4, 1, 16)
    grid = (B, NC)
    return pl.pallas_call(
        _phase_a_body,
        grid=grid,
        in_specs=[
            pl.BlockSpec((1, 1, 16), lambda b, c: (b, 0, 0)),
            pl.BlockSpec((1, 50, 8), lambda b, c: (b, 0, 0)),
            pl.BlockSpec((1, CB, H, W), lambda b, c: (b, c, 0, 0)),
        ],
        out_specs=[
            pl.BlockSpec((1, 1, 128), lambda b, c: (b, 0, 0)),
            pl.BlockSpec((1, 1, 128), lambda b, c: (b, 0, 0)),
            pl.BlockSpec((1, 1, 16), lambda b, c: (b, 0, 0)),
        ],
        out_shape=[
            jax.ShapeDtypeStruct((B, 1, 128), jnp.int32),
            jax.ShapeDtypeStruct((B, 1, 128), jnp.int32),
            jax.ShapeDtypeStruct((B, 1, 16), jnp.int32),
        ],
        scratch_shapes=[pltpu.VMEM((H, W), jnp.float32)],
        compiler_params=pltpu.CompilerParams(
            dimension_semantics=("arbitrary", "arbitrary")),
    )(perm, gt_boxes, x)


NV = 80 + 60 + 120
VPT = 17
N_GEO = 80
N_ST = 60
N_AMB = 120
AMB_PER_SC = 60
ROWS_PER_TILE = 4
TROWS = 3 * 16 * VPT
BIAS_INVALID = -1e37


_SLOT_MAP = np.minimum(
    (np.arange(16)[:, None] * VPT + np.arange(32)[None, :]), NV - 1
).astype(np.int32)


def _phase_b(x1d, base_tbl, xpos_tbl, bias_tbl):
    mesh = plsc.VectorSubcoreMesh(core_axis_name="c", subcore_axis_name="s")

    @functools.partial(
        pl.kernel,
        out_type=jax.ShapeDtypeStruct((32, 16), jnp.float32),
        mesh=mesh,
        compiler_params=pltpu.CompilerParams(
            needs_layout_passes=False, use_tc_tiling_on_sc=True),
        scratch_types=[
            pltpu.VMEM((32,), jnp.int32),
            pltpu.VMEM((32,), jnp.int32),
            pltpu.VMEM((32,), jnp.float32),
            pltpu.VMEM((2 * VPT, 128), jnp.int32),
            pltpu.VMEM((3 * VPT, 128), jnp.float32),
            pltpu.VMEM((2, 128, 128), jnp.float32),
            pltpu.VMEM((3 * (N_GEO + N_ST), 128), jnp.float32),
            pltpu.VMEM((3 * ROWS_PER_TILE, 128), jnp.float32),
            pltpu.VMEM((16,), jnp.float32),
            pltpu.VMEM_SHARED((TROWS, 128), jnp.float32),
            pltpu.SMEM((24,), jnp.int32),
            pltpu.SMEM((24,), jnp.int32),
            pltpu.SMEM((24,), jnp.float32),
            pltpu.SemaphoreType.DMA,
        ],
    )
    def body(x_hbm, base_hbm, xpos_hbm, bias_hbm, out_hbm,
             base_v, xpos_v, bias_v, gidx, gbuf, rawbuf, tgt, myamb, obuf,
             table, sm_base, sm_x, sm_bias, sem):
        x2 = x_hbm.reshape(B * C * H, W)
        s = lax.axis_index("s")
        c = lax.axis_index("c")
        wid = s * 2 + c

        pltpu.sync_copy(base_hbm.at[s], base_v)
        pltpu.sync_copy(bias_hbm.at[s], bias_v)
        pltpu.sync_copy(xpos_hbm.at[s], xpos_v)
        iota16 = lax.broadcasted_iota(jnp.int32, (16,), 0)
        zeros16 = jnp.zeros((16,), jnp.float32)
        v0 = s * VPT
        bvecs = [base_v[pl.ds(0, 16)], base_v[pl.ds(16, 16)]]
        xvecs = [xpos_v[pl.ds(0, 16)], xpos_v[pl.ds(16, 16)]]
        fvecs = [bias_v[pl.ds(0, 16)], bias_v[pl.ds(16, 16)]]

        for i in range(VPT):
            sm_base[i] = bvecs[i // 16][i % 16]
            sm_x[i] = xvecs[i // 16][i % 16]
            sm_bias[i] = fvecs[i // 16][i % 16]

        def _build(i, carry):
            base = sm_base[i]
            for q in range(16):
                gidx[2 * i + q // 8, pl.ds((q % 8) * 16, 16)] = (
                    base + H * (q * 16 + iota16))
            return carry

        lax.fori_loop(0, VPT, _build, 0)

        def _fire(hh):
            xa = (sm_x[hh >> 1] >> 7) * 128
            pltpu.async_copy(
                x2.at[gidx.at[hh], pl.ds(xa, 128)], rawbuf.at[hh & 1], sem)

        def _gstep(hh, carry):
            @pl.when(hh + 1 < 2 * VPT)
            def _():
                _fire(hh + 1)

            i = hh >> 1
            half = hh & 1
            xa = (sm_x[i] >> 7) * 128
            pltpu.make_async_copy(
                x2.at[gidx.at[hh], pl.ds(xa, 128)], rawbuf.at[hh & 1],
                sem).wait()
            lane = jnp.broadcast_to(sm_x[i] & 127, (16,))
            rb = rawbuf.at[hh & 1]
            for g in range(8):
                ch = plsc.load_gather(rb, [g * 16 + iota16, lane])
                gbuf[3 * i + half, pl.ds(g * 16, 16)] = ch
            return carry

        _fire(0)
        lax.fori_loop(0, 2 * VPT, _gstep, 0)

        def _norm(i, carry):
            chunks = []
            acc = None
            for q in range(16):
                ch = gbuf[3 * i + q // 8, pl.ds((q % 8) * 16, 16)]
                chunks.append(ch)
                acc = ch * ch if acc is None else acc + ch * ch
            ssq = jnp.sum(acc)
            sv = jnp.broadcast_to(ssq, (16,))
            yi = jnp.int32(0x5F3759DF) - (plsc.bitcast(sv, jnp.int32) >> 1)
            y = plsc.bitcast(yi, jnp.float32)
            for _ in range(3):
                y = y * (1.5 - 0.5 * sv * y * y)
            inv = jnp.where(sv > 1e-24, y, jnp.full((16,), 1e12, jnp.float32))
            for q in range(16):
                gbuf[3 * i + q // 8, pl.ds((q % 8) * 16, 16)] = (
                    chunks[q] * inv)
            brow = sm_bias[i]
            gbuf[3 * i + 2, pl.ds(0, 16)] = jnp.where(
                iota16 == 0, brow, zeros16)
            for q in range(1, 8):
                gbuf[3 * i + 2, pl.ds(q * 16, 16)] = zeros16
            return carry

        lax.fori_loop(0, VPT, _norm, 0)

        pltpu.sync_copy(gbuf, table.at[pl.ds(v0 * 3, 3 * VPT)])
        plsc.subcore_barrier()
        pltpu.sync_copy(table.at[pl.ds(0, 3 * (N_GEO + N_ST))], tgt)
        amb_row0 = N_GEO + N_ST + c * AMB_PER_SC + s * ROWS_PER_TILE
        pltpu.sync_copy(
            table.at[pl.ds(amb_row0 * 3, 3 * ROWS_PER_TILE)], myamb)

        total = jnp.float32(0.0)
        inv_temp = jnp.float32(float(np.float32(1.0) / np.float32(TEMP)))
        for j in range(ROWS_PER_TILE):
            arow = [myamb[3 * j + q // 8, pl.ds((q % 8) * 16, 16)]
                    for q in range(16)]
            arow.append(myamb[3 * j + 2, pl.ds(0, 16)])

            def _sim(t):
                acc = None
                for q in range(17):
                    row = 3 * t + (q // 8 if q < 16 else 2)
                    off = (q % 8) * 16 if q < 16 else 0
                    tc = tgt[row, pl.ds(off, 16)]
                    acc = arow[q] * tc if acc is None else acc + arow[q] * tc
                return jnp.sum(acc) * inv_temp

            def _max(t, m):
                return jnp.maximum(m, _sim(t))

            mp = lax.fori_loop(0, N_GEO, _max, jnp.float32(-3.4e38))
            mn = lax.fori_loop(N_GEO, N_GEO + N_ST, _max,
                               jnp.float32(-3.4e38))
            term = jnp.maximum(jnp.float32(MARGIN) + mn - mp,
                               jnp.float32(0.0))
            in_range = s * ROWS_PER_TILE + j < AMB_PER_SC
            total = total + jnp.where(in_range, term, jnp.float32(0.0))

        obuf[...] = jnp.broadcast_to(total, (16,))
        pltpu.sync_copy(obuf, out_hbm.at[wid])

    return body(x1d, base_tbl, xpos_tbl, bias_tbl)


def kernel(aligned_loc_feature, gt_boxes):
    geo, gv, st = _phase_a(aligned_loc_feature, gt_boxes)
    geo2 = geo.reshape(B, 128)[:, :TOPK_GEO]
    gv2 = gv.reshape(B, 128)[:, :TOPK_GEO]
    st2 = st.reshape(B, 16)[:, :TOPK_STATIC]
    amb2 = jnp.asarray(_AMB_IDX)
    bvals = jnp.concatenate([
        jnp.repeat(jnp.arange(B, dtype=jnp.int32), TOPK_GEO),
        jnp.repeat(jnp.arange(B, dtype=jnp.int32), TOPK_STATIC),
        jnp.repeat(jnp.arange(B, dtype=jnp.int32), NUM_AMB)])
    pvals = jnp.concatenate(
        [geo2.reshape(-1), st2.reshape(-1), amb2.reshape(-1)])
    py, px = pvals >> 8, pvals & 255
    base_flat = bvals * (C * H) + py
    xpos_flat = px
    bias_flat = jnp.concatenate([
        jnp.where(gv2.reshape(-1) != 0, 0.0, BIAS_INVALID)
        .astype(jnp.float32),
        jnp.zeros((N_ST,), jnp.float32),
        jnp.ones((N_AMB,), jnp.float32)])
    slot = jnp.asarray(_SLOT_MAP)
    partials = _phase_b(aligned_loc_feature, base_flat[slot],
                        xpos_flat[slot], bias_flat[slot])
    return jnp.sum(partials[:, 0]) / jnp.float32(N_AMB)


# speedup vs baseline: 1.6777x; 1.0241x over previous
import functools

import jax
import jax.numpy as jnp
import numpy as np
from jax import lax
from jax.experimental import pallas as pl
from jax.experimental.pallas import tpu as pltpu
from jax.experimental.pallas import tpu_sc as plsc

B, C, H, W = 4, 256, 256, 256
TOPK_GEO = 20
TOPK_STATIC = 15
NUM_AMB = 30
TEMP = 0.1
MARGIN = 0.3
PCR0, PCR1 = -59.9, -59.9
BEGIN_W = 119.8
BEGIN_H = 119.8
NEG_INF = float("-inf")

CB = 32
NC = C // CB

_AMB_IDX = np.array([
    [42684, 39799, 17101, 26684, 45502, 49427, 52287, 15840, 51402, 10298,
     15388, 5700, 62806, 48744, 54950, 1748, 30316, 65134, 23053, 13198,
     40558, 26643, 43274, 16075, 23612, 1587, 29516, 8934, 60384, 4667],
    [8159, 21315, 62515, 17158, 55309, 59725, 2179, 36669, 1658, 62896,
     9308, 39123, 42076, 23448, 35406, 13534, 51694, 37155, 53091, 24021,
     19512, 28969, 18536, 8640, 62520, 51289, 1823, 10367, 53219, 3871],
    [11764, 16683, 42054, 43176, 18925, 52470, 33349, 61666, 38626, 7651,
     8719, 61782, 25143, 31664, 42585, 38596, 65445, 34154, 60634, 18834,
     37711, 21348, 30494, 37663, 56142, 1625, 24231, 53445, 7680, 31046],
    [1614, 28512, 63057, 22698, 20162, 35211, 47616, 17187, 64995, 28178,
     36068, 40764, 24733, 1009, 15346, 26315, 6708, 34707, 37933, 19975,
     47948, 45082, 17151, 32451, 5007, 5526, 40990, 1517, 9641, 1470]],
    dtype=np.int32)

_STATIC_PERM = np.array([
    [35, 6, 30, 28, 33, 1, 2, 37, 45, 17, 32, 48, 14, 4, 7],
    [11, 3, 18, 10, 44, 16, 21, 48, 17, 35, 2, 23, 33, 43, 0],
    [14, 13, 15, 7, 17, 11, 32, 44, 35, 1, 36, 28, 31, 2, 9],
    [28, 0, 47, 27, 48, 45, 29, 15, 1, 25, 16, 13, 42, 46, 19]],
    dtype=np.int32)
_STATIC_PERM16 = np.concatenate(
    [_STATIC_PERM, np.full((4, 1), -1, np.int32)], axis=1)


def _phase_a_body(perm_ref, gt_ref, x_ref, geo_ref, gv_ref, st_ref, acc_ref):
    c = pl.program_id(1)
    blk = x_ref[0]
    part = jnp.sum(blk * blk, axis=0)

    @pl.when(c == 0)
    def _init():
        acc_ref[...] = part

    @pl.when(c > 0)
    def _acc():
        acc_ref[...] = acc_ref[...] + part

    @pl.when(c == NC - 1)
    def _finish():
        inten = jnp.sqrt(acc_ref[...])
        ninf = jnp.float32(NEG_INF)
        pad_row = jnp.full((1, W), ninf, jnp.float32)
        up = jnp.concatenate([inten[1:, :], pad_row], axis=0)
        dn = jnp.concatenate([pad_row, inten[:-1, :]], axis=0)
        rowm = jnp.maximum(inten, jnp.maximum(up, dn))
        pad_col = jnp.full((H, 1), ninf, jnp.float32)
        lf = jnp.concatenate([rowm[:, 1:], pad_col], axis=1)
        rt = jnp.concatenate([pad_col, rowm[:, :-1]], axis=1)
        pooled = jnp.maximum(rowm, jnp.maximum(lf, rt))
        mask = inten == pooled
        cand0 = jnp.where(mask, inten, ninf)
        flat = (lax.broadcasted_iota(jnp.int32, (H, W), 0) * W
                + lax.broadcasted_iota(jnp.int32, (H, W), 1))
        lane = lax.broadcasted_iota(jnp.int32, (1, 128), 1)

        def topk_step(k, carry):
            cand, idxv, gvv = carry
            m = jnp.max(cand)
            sel = jnp.where(cand == m, flat, jnp.int32(1 << 30))
            i = jnp.min(sel)
            idxv = jnp.where(lane == k, i, idxv)
            gvv = jnp.where(lane == k,
                            jnp.where(m > ninf, 1, 0), gvv)
            cand = jnp.where(flat == i, ninf, cand)
            return cand, idxv, gvv

        _, idxv, gvv = lax.fori_loop(
            0, TOPK_GEO, topk_step,
            (cand0, jnp.zeros((1, 128), jnp.int32),
             jnp.zeros((1, 128), jnp.int32)))
        geo_ref[0] = idxv
        gv_ref[0] = gvv

        boxes = gt_ref[0]
        n = boxes.shape[0]
        kcol = (boxes[:, 7:8] != 0).astype(jnp.float32)
        tri = (lax.broadcasted_iota(jnp.int32, (n, n), 0)
               > lax.broadcasted_iota(jnp.int32, (n, n), 1)
               ).astype(jnp.float32)
        ones_before = jnp.dot(tri, kcol,
                              preferred_element_type=jnp.float32)
        zeros_before = jnp.dot(tri, 1.0 - kcol,
                               preferred_element_type=jnp.float32)
        nz = jnp.sum(1.0 - kcol)
        pos = jnp.where(kcol > 0, nz + ones_before, zeros_before)
        targets = perm_ref[0].astype(jnp.float32)
        eq = (pos == targets).astype(jnp.float32)
        bx = jnp.sum(eq * boxes[:, 0:1], axis=0, keepdims=True)
        by = jnp.sum(eq * boxes[:, 1:2], axis=0, keepdims=True)
        cx = jnp.clip((bx - PCR0) / BEGIN_W * W, 0, W - 1).astype(jnp.int32)
        cy = jnp.clip((by - PCR1) / BEGIN_H * H, 0, H - 1).astype(jnp.int32)
        st_ref[0] = cy * W + cx


def _phase_a(x, gt_boxes):
    perm = jnp.asarray(_STATIC_PERM16).reshape(4, 1, 16)
    grid = (B, NC)
    return pl.pallas_call(
        _phase_a_body,
        grid=grid,
        in_specs=[
            pl.BlockSpec((1, 1, 16), lambda b, c: (b, 0, 0)),
            pl.BlockSpec((1, 50, 8), lambda b, c: (b, 0, 0)),
            pl.BlockSpec((1, CB, H, W), lambda b, c: (b, c, 0, 0)),
        ],
        out_specs=[
            pl.BlockSpec((1, 1, 128), lambda b, c: (b, 0, 0)),
            pl.BlockSpec((1, 1, 128), lambda b, c: (b, 0, 0)),
            pl.BlockSpec((1, 1, 16), lambda b, c: (b, 0, 0)),
        ],
        out_shape=[
            jax.ShapeDtypeStruct((B, 1, 128), jnp.int32),
            jax.ShapeDtypeStruct((B, 1, 128), jnp.int32),
            jax.ShapeDtypeStruct((B, 1, 16), jnp.int32),
        ],
        scratch_shapes=[pltpu.VMEM((H, W), jnp.float32)],
        compiler_params=pltpu.CompilerParams(
            dimension_semantics=("arbitrary", "arbitrary")),
    )(perm, gt_boxes, x)


NV = 80 + 60 + 120
N_GEO = 80
N_ST = 60
N_TGT = N_GEO + N_ST
N_AMB = 120
AMB_PER_SC = 60
AVPT = 4
VPT = 9
ROWS_PER_TILE = 4
TROWS = 3 * 16 * VPT
BIAS_INVALID = -1e37

_SLOT_T = np.minimum(
    (np.arange(16)[:, None] * VPT + np.arange(32)[None, :]), N_TGT - 1
).astype(np.int32)
_SLOT_A = np.minimum(
    np.arange(2)[:, None, None] * AMB_PER_SC
    + np.arange(16)[None, :, None] * AVPT
    + np.arange(32)[None, None, :], N_AMB - 1).astype(np.int32)
_AMB_FLAT = _AMB_IDX.reshape(-1)
_AMB_B = (np.arange(N_AMB) // NUM_AMB).astype(np.int32)
_A_BASE = (_AMB_B * (C * H) + (_AMB_FLAT >> 8))[_SLOT_A]
_A_XPOS = (_AMB_FLAT & 255)[_SLOT_A]


def _sc_gather(x2, gidx, gbuf, rawbuf, sm_base, sm_x, sem, nvec, iota16):
    def _build(i, carry):
        base = sm_base[i]
        for q in range(16):
            gidx[2 * i + q // 8, pl.ds((q % 8) * 16, 16)] = (
                base + H * (q * 16 + iota16))
        return carry

    lax.fori_loop(0, nvec, _build, 0)

    def _fire(hh):
        xa = (sm_x[hh >> 1] >> 7) * 128
        pltpu.async_copy(
            x2.at[gidx.at[hh], pl.ds(xa, 128)], rawbuf.at[hh & 1], sem)

    def _gstep(hh, carry):
        @pl.when(hh + 1 < 2 * nvec)
        def _():
            _fire(hh + 1)

        i = hh >> 1
        half = hh & 1
        xa = (sm_x[i] >> 7) * 128
        pltpu.make_async_copy(
            x2.at[gidx.at[hh], pl.ds(xa, 128)], rawbuf.at[hh & 1],
            sem).wait()
        lane = jnp.broadcast_to(sm_x[i] & 127, (16,))
        rb = rawbuf.at[hh & 1]
        for g in range(8):
            ch = plsc.load_gather(rb, [g * 16 + iota16, lane])
            gbuf[3 * i + half, pl.ds(g * 16, 16)] = ch
        return carry

    _fire(0)
    lax.fori_loop(0, 2 * nvec, _gstep, 0)


def _sc_normalize(gbuf, sm_bias, nvec, iota16, zeros16, const_bias=None):
    def _norm(i, carry):
        chunks = []
        acc = None
        for q in range(16):
            ch = gbuf[3 * i + q // 8, pl.ds((q % 8) * 16, 16)]
            chunks.append(ch)
            acc = ch * ch if acc is None else acc + ch * ch
        ssq = jnp.sum(acc)
        sv = jnp.broadcast_to(ssq, (16,))
        yi = jnp.int32(0x5F3759DF) - (plsc.bitcast(sv, jnp.int32) >> 1)
        y = plsc.bitcast(yi, jnp.float32)
        for _ in range(3):
            y = y * (1.5 - 0.5 * sv * y * y)
        inv = jnp.where(sv > 1e-24, y, jnp.full((16,), 1e12, jnp.float32))
        for q in range(16):
            gbuf[3 * i + q // 8, pl.ds((q % 8) * 16, 16)] = chunks[q] * inv
        brow = jnp.float32(const_bias) if const_bias is not None else sm_bias[i]
        gbuf[3 * i + 2, pl.ds(0, 16)] = jnp.where(iota16 == 0, brow, zeros16)
        for q in range(1, 8):
            gbuf[3 * i + 2, pl.ds(q * 16, 16)] = zeros16
        return carry

    lax.fori_loop(0, nvec, _norm, 0)


def _phase_b1(x, abase, axpos):
    mesh = plsc.VectorSubcoreMesh(core_axis_name="c", subcore_axis_name="s")

    @functools.partial(
        pl.kernel,
        out_type=jax.ShapeDtypeStruct((2, 16 * 16, 128), jnp.float32),
        mesh=mesh,
        compiler_params=pltpu.CompilerParams(
            needs_layout_passes=False, use_tc_tiling_on_sc=True),
        scratch_types=[
            pltpu.VMEM((32,), jnp.int32),
            pltpu.VMEM((32,), jnp.int32),
            pltpu.VMEM((2 * AVPT, 128), jnp.int32),
            pltpu.VMEM((16, 128), jnp.float32),
            pltpu.VMEM((2, 128, 128), jnp.float32),
            pltpu.SMEM((8,), jnp.int32),
            pltpu.SMEM((8,), jnp.int32),
            pltpu.SemaphoreType.DMA,
        ],
    )
    def body(x_hbm, base_hbm, xpos_hbm, out_hbm,
             base_v, xpos_v, gidx, gbuf, rawbuf, sm_base, sm_x, sem):
        x2 = x_hbm.reshape(B * C * H, W)
        s = lax.axis_index("s")
        c = lax.axis_index("c")
        pltpu.sync_copy(base_hbm.at[c, s], base_v)
        pltpu.sync_copy(xpos_hbm.at[c, s], xpos_v)
        iota16 = lax.broadcasted_iota(jnp.int32, (16,), 0)
        zeros16 = jnp.zeros((16,), jnp.float32)
        bv = base_v[pl.ds(0, 16)]
        xv = xpos_v[pl.ds(0, 16)]
        for i in range(AVPT):
            sm_base[i] = bv[i]
            sm_x[i] = xv[i]
        _sc_gather(x2, gidx, gbuf, rawbuf, sm_base, sm_x, sem, AVPT, iota16)
        _sc_normalize(gbuf, None, AVPT, iota16, zeros16, const_bias=1.0)
        pltpu.sync_copy(gbuf, out_hbm.at[c, pl.ds(s * 16, 16)])

    return body(x, abase, axpos)


def _phase_b2(x, tbase, txpos, tbias, ambfeat):
    mesh = plsc.VectorSubcoreMesh(core_axis_name="c", subcore_axis_name="s")

    @functools.partial(
        pl.kernel,
        out_type=jax.ShapeDtypeStruct((32, 16), jnp.float32),
        mesh=mesh,
        compiler_params=pltpu.CompilerParams(
            needs_layout_passes=False, use_tc_tiling_on_sc=True),
        scratch_types=[
            pltpu.VMEM((32,), jnp.int32),
            pltpu.VMEM((32,), jnp.int32),
            pltpu.VMEM((32,), jnp.float32),
            pltpu.VMEM((2 * VPT, 128), jnp.int32),
            pltpu.VMEM((3 * VPT, 128), jnp.float32),
            pltpu.VMEM((2, 128, 128), jnp.float32),
            pltpu.VMEM((3 * N_TGT, 128), jnp.float32),
            pltpu.VMEM((16, 128), jnp.float32),
            pltpu.VMEM((16,), jnp.float32),
            pltpu.VMEM_SHARED((TROWS, 128), jnp.float32),
            pltpu.SMEM((16,), jnp.int32),
            pltpu.SMEM((16,), jnp.int32),
            pltpu.SMEM((16,), jnp.float32),
            pltpu.SemaphoreType.DMA,
        ],
    )
    def body(x_hbm, base_hbm, xpos_hbm, bias_hbm, amb_hbm, out_hbm,
             base_v, xpos_v, bias_v, gidx, gbuf, rawbuf, tgt, myamb, obuf,
             table, sm_base, sm_x, sm_bias, sem):
        x2 = x_hbm.reshape(B * C * H, W)
        s = lax.axis_index("s")
        c = lax.axis_index("c")
        wid = s * 2 + c

        pltpu.sync_copy(base_hbm.at[s], base_v)
        pltpu.sync_copy(xpos_hbm.at[s], xpos_v)
        pltpu.sync_copy(bias_hbm.at[s], bias_v)
        amb_v0 = c * AMB_PER_SC + s * ROWS_PER_TILE
        pltpu.sync_copy(amb_hbm.at[c, pl.ds(s * 16, 16)], myamb)
        iota16 = lax.broadcasted_iota(jnp.int32, (16,), 0)
        zeros16 = jnp.zeros((16,), jnp.float32)
        bv = base_v[pl.ds(0, 16)]
        xv = xpos_v[pl.ds(0, 16)]
        fv = bias_v[pl.ds(0, 16)]
        for i in range(VPT):
            sm_base[i] = bv[i]
            sm_x[i] = xv[i]
            sm_bias[i] = fv[i]
        _sc_gather(x2, gidx, gbuf, rawbuf, sm_base, sm_x, sem, VPT, iota16)
        _sc_normalize(gbuf, sm_bias, VPT, iota16, zeros16)

        pltpu.sync_copy(gbuf, table.at[pl.ds(s * 3 * VPT, 3 * VPT)])
        plsc.subcore_barrier()
        pltpu.sync_copy(table.at[pl.ds(0, 3 * N_TGT)], tgt)

        total = jnp.float32(0.0)
        inv_temp = jnp.float32(float(np.float32(1.0) / np.float32(TEMP)))
        for j in range(ROWS_PER_TILE):
            arow = [myamb[3 * j + q // 8, pl.ds((q % 8) * 16, 16)]
                    for q in range(16)]
            arow.append(myamb[3 * j + 2, pl.ds(0, 16)])

            def _sim(t):
                acc = None
                for q in range(17):
                    row = 3 * t + (q // 8 if q < 16 else 2)
                    off = (q % 8) * 16 if q < 16 else 0
                    tc = tgt[row, pl.ds(off, 16)]
                    acc = arow[q] * tc if acc is None else acc + arow[q] * tc
                return jnp.sum(acc) * inv_temp

            def _max(t, m):
                return jnp.maximum(m, _sim(t))

            mp = lax.fori_loop(0, N_GEO, _max, jnp.float32(-3.4e38))
            mn = lax.fori_loop(N_GEO, N_TGT, _max, jnp.float32(-3.4e38))
            term = jnp.maximum(jnp.float32(MARGIN) + mn - mp,
                               jnp.float32(0.0))
            in_range = s * ROWS_PER_TILE + j < AMB_PER_SC
            total = total + jnp.where(in_range, term, jnp.float32(0.0))

        obuf[...] = jnp.broadcast_to(total, (16,))
        pltpu.sync_copy(obuf, out_hbm.at[wid])

    return body(x, tbase, txpos, tbias, ambfeat)


def kernel(aligned_loc_feature, gt_boxes):
    geo, gv, st = _phase_a(aligned_loc_feature, gt_boxes)
    ambfeat = _phase_b1(aligned_loc_feature, jnp.asarray(_A_BASE),
                        jnp.asarray(_A_XPOS))
    geo2 = geo.reshape(B, 128)[:, :TOPK_GEO]
    gv2 = gv.reshape(B, 128)[:, :TOPK_GEO]
    st2 = st.reshape(B, 16)[:, :TOPK_STATIC]
    bvals = jnp.concatenate([
        jnp.repeat(jnp.arange(B, dtype=jnp.int32), TOPK_GEO),
        jnp.repeat(jnp.arange(B, dtype=jnp.int32), TOPK_STATIC)])
    pvals = jnp.concatenate([geo2.reshape(-1), st2.reshape(-1)])
    base_flat = bvals * (C * H) + (pvals >> 8)
    xpos_flat = pvals & 255
    bias_flat = jnp.concatenate([
        jnp.where(gv2.reshape(-1) != 0, 0.0, BIAS_INVALID)
        .astype(jnp.float32),
        jnp.zeros((N_ST,), jnp.float32)])
    slot = jnp.asarray(_SLOT_T)
    partials = _phase_b2(aligned_loc_feature, base_flat[slot],
                         xpos_flat[slot], bias_flat[slot], ambfeat)
    return jnp.sum(partials[:, 0]) / jnp.float32(N_AMB)
